# baseline (speedup 1.0000x reference)
import jax
import jax.numpy as jnp
import numpy as np
from jax.experimental import pallas as pl
from jax.experimental.pallas import tpu as pltpu

_HEAD_ORDER = ['role_select', 'settler', 'builder', 'mayor', 'craftsman', 'trader', 'captain']
_HEAD_HIDDEN = [512, 256, 512, 512, 128, 256, 512]
_PHASE_TO_HEADIDX = np.array([1, 3, 2, 4, 5, 6, 6, 0, 0], dtype=np.int32)
_OFFS = np.concatenate([[0], np.cumsum(_HEAD_HIDDEN)])
_HSUM = int(_OFFS[-1])
_H = 512
_ACT = 200
_OBS = 210


def _bdot16(a, b):
    return jax.lax.dot(a, b, preferred_element_type=jnp.float32)


def _bdot(a, b):
    return jax.lax.dot(a.astype(jnp.bfloat16), b, preferred_element_type=jnp.float32)


def _normalize(x, eps=1e-5):
    m = jnp.mean(x, axis=-1, keepdims=True)
    v = jnp.mean(x * x, axis=-1, keepdims=True) - m * m
    return (x - m) * jax.lax.rsqrt(v + eps)


def _fused_body(*refs):
    (x_ref, ph_ref, act_ref, amask_ref,
     pe_tab_ref, ew_ref, be_ref, ge_ref, bee_ref,
     b1g, b1b, b1w1, b1b1, b1w2, b1b2,
     b2g, b2b, b2w1, b2b1, b2w2, b2b2,
     b3g, b3b, b3w1, b3b1, b3w2, b3b2,
     cg, cb, cw1, cb1, cw2, cb2,
     h0g, h0b, h0w1, h0b1, h0w2, h0b2,
     h1g, h1b, h1w1, h1b1, h1w2, h1b2,
     h2g, h2b, h2w1, h2b1, h2w2, h2b2,
     h3g, h3b, h3w1, h3b1, h3w2, h3b2,
     h4g, h4b, h4w1, h4b1, h4w2, h4b2,
     h5g, h5b, h5w1, h5b1, h5w2, h5b2,
     h6g, h6b, h6w1, h6b1, h6w2, h6b2,
     p2h_ref,
     logp_ref, ent_ref, val_ref,
     wxs, wps,
     bw1s_1, bb1s_1, bw2s_1,
     bw1s_2, bb1s_2, bw2s_2,
     bw1s_3, bb1s_3, bw2s_3,
     cw1s, cb1s, cw2s,
     hw1s, hb1s, hw2s, hb2s) = refs

    f32 = jnp.float32
    bf16 = jnp.bfloat16
    blk = x_ref.shape[0]

    @pl.when(pl.program_id(0) == 0)
    def _prep():
        ew = ew_ref[...]
        wxs[...] = ew[:_OBS].astype(bf16)
        wps[...] = ew[_OBS:].astype(bf16)
        for (g_r, b_r, w1_r, b1_r, w2_r, w1_o, b1_o, w2_o) in (
            (b1g, b1b, b1w1, b1b1, b1w2, bw1s_1, bb1s_1, bw2s_1),
            (b2g, b2b, b2w1, b2b1, b2w2, bw1s_2, bb1s_2, bw2s_2),
            (b3g, b3b, b3w1, b3b1, b3w2, bw1s_3, bb1s_3, bw2s_3),
        ):
            w1 = w1_r[...]
            w1_o[...] = (g_r[...][:, None] * w1).astype(bf16)
            b1_o[...] = b1_r[...][None, :] + jnp.dot(b_r[...][None, :], w1)
            w2_o[...] = w2_r[...].astype(bf16)
        w1 = cw1[...]
        cw1s[...] = (cg[...][:, None] * w1).astype(bf16)
        cb1s[...] = cb1[...][None, :] + jnp.dot(cb[...][None, :], w1)
        cw2s[...] = cw2[...].astype(bf16)
        heads = (
            (h0g, h0b, h0w1, h0b1, h0w2, h0b2),
            (h1g, h1b, h1w1, h1b1, h1w2, h1b2),
            (h2g, h2b, h2w1, h2b1, h2w2, h2b2),
            (h3g, h3b, h3w1, h3b1, h3w2, h3b2),
            (h4g, h4b, h4w1, h4b1, h4w2, h4b2),
            (h5g, h5b, h5w1, h5b1, h5w2, h5b2),
            (h6g, h6b, h6w1, h6b1, h6w2, h6b2),
        )
        for k, (g_r, b_r, w1_r, b1_r, w2_r, b2_r) in enumerate(heads):
            off, hh = int(_OFFS[k]), _HEAD_HIDDEN[k]
            w1 = w1_r[...]
            hw1s[:, off:off + hh] = (g_r[...][:, None] * w1).astype(bf16)
            hb1s[:, off:off + hh] = b1_r[...][None, :] + jnp.dot(b_r[...][None, :], w1)
            hw2s[off:off + hh, :] = w2_r[...].astype(bf16)
            hb2s[k:k + 1, :] = b2_r[...][None, :]
        hb2s[7:8, :] = jnp.zeros((1, _ACT), f32)

    ph = ph_ref[...]
    iota9 = jax.lax.broadcasted_iota(jnp.int32, (blk, 9), 1)
    oh9 = (ph == iota9).astype(f32)
    pe = jnp.dot(oh9, pe_tab_ref[...])

    u = _bdot(x_ref[...], wxs[...]) + _bdot(pe, wps[...]) + be_ref[...]
    h = jax.nn.relu(_normalize(u) * ge_ref[...] + bee_ref[...])

    for (w1, b1, w2, b2) in (
        (bw1s_1, bb1s_1, bw2s_1, b1b2),
        (bw1s_2, bb1s_2, bw2s_2, b2b2),
        (bw1s_3, bb1s_3, bw2s_3, b3b2),
    ):
        t = _normalize(h).astype(bf16)
        t = jax.nn.relu(_bdot16(t, w1[...]) + b1[...])
        t = jax.nn.relu(_bdot(t, w2[...]) + b2[...])
        h = h + t

    nrm = _normalize(h).astype(bf16)

    v = jax.nn.relu(_bdot16(nrm, cw1s[...]) + cb1s[...])
    val_ref[...] = _bdot(v, cw2s[...]) + cb2[...]

    h1 = _bdot16(nrm, hw1s[...]) + hb1s[...]

    hid = jnp.dot(oh9, p2h_ref[...]).astype(jnp.int32)
    cols = jax.lax.broadcasted_iota(jnp.int32, (1, _HSUM), 1)
    seg = jnp.zeros((1, _HSUM), jnp.int32)
    for off in _OFFS[1:-1]:
        seg = seg + (cols >= int(off)).astype(jnp.int32)
    h1m = jnp.where(seg == hid, jax.nn.relu(h1).astype(bf16), jnp.bfloat16(0))

    logits = _bdot16(h1m, hw2s[...])
    iota8 = jax.lax.broadcasted_iota(jnp.int32, (blk, 8), 1)
    oh8 = (iota8 == hid).astype(f32)
    logits = logits + jnp.dot(oh8, hb2s[...])

    masked = jnp.where(amask_ref[...] > 0.5, logits, f32(-1e8))
    mx = jnp.max(masked, axis=-1, keepdims=True)
    z = masked - mx
    ez = jnp.exp(z)
    s = jnp.sum(ez, axis=-1, keepdims=True)
    logp = z - jnp.log(s)

    act = act_ref[...]
    iota_a = jax.lax.broadcasted_iota(jnp.int32, (blk, logits.shape[1]), 1)
    oh_a = (act == iota_a).astype(f32)
    logp_ref[...] = jnp.sum(logp * oh_a, axis=-1, keepdims=True)
    probs = ez / s
    ent_ref[...] = -jnp.sum(probs * logp, axis=-1, keepdims=True)


@jax.jit
def _run(x, action_mask, phase_ids, action, params):
    B, OBS = x.shape
    ACT = action_mask.shape[1]
    H = _H
    PE = params['phase_embed'].shape[1]
    BLK = 1024
    nb = B // BLK
    bf16 = jnp.bfloat16
    f32 = jnp.float32

    e = params['embed']
    ph2 = phase_ids.astype(jnp.int32).reshape(B, 1)
    act2 = action.astype(jnp.int32).reshape(B, 1)

    row_spec = lambda w: pl.BlockSpec((BLK, w), lambda i: (i, 0))
    full = lambda *shape: pl.BlockSpec(shape, lambda i: (0,) * len(shape))

    ins = [x, ph2, act2, action_mask,
           params['phase_embed'], e['W'], e['b'], e['g'], e['be']]
    in_specs = [row_spec(OBS), row_spec(1), row_spec(1), row_spec(ACT),
                full(9, PE), full(OBS + 16, H), full(H), full(H), full(H)]
    for b in params['blocks']:
        ins += [b['g'], b['be'], b['W1'], b['b1'], b['W2'], b['b2']]
        in_specs += [full(H), full(H), full(H, H), full(H), full(H, H), full(H)]
    c = params['critic']
    ins += [c['g'], c['be'], c['W1'], c['b1'], c['W2'], c['b2']]
    in_specs += [full(H), full(H), full(H, H), full(H), full(H, 1), full(1)]
    for n, hh in zip(_HEAD_ORDER, _HEAD_HIDDEN):
        hp = params['heads'][n]
        ins += [hp['g'], hp['be'], hp['W1'], hp['b1'], hp['W2'], hp['b2']]
        in_specs += [full(H), full(H), full(H, hh), full(hh), full(hh, ACT), full(ACT)]
    ins += [jnp.asarray(_PHASE_TO_HEADIDX.astype(np.float32)[:, None])]
    in_specs += [full(9, 1)]

    scratch = [
        pltpu.VMEM((OBS, H), bf16), pltpu.VMEM((16, H), bf16),
    ]
    for _ in range(3):
        scratch += [pltpu.VMEM((H, H), bf16), pltpu.VMEM((1, H), f32),
                    pltpu.VMEM((H, H), bf16)]
    scratch += [pltpu.VMEM((H, H), bf16), pltpu.VMEM((1, H), f32),
                pltpu.VMEM((H, 1), bf16)]
    scratch += [pltpu.VMEM((H, _HSUM), bf16), pltpu.VMEM((1, _HSUM), f32),
                pltpu.VMEM((_HSUM, ACT), bf16), pltpu.VMEM((8, ACT), f32)]

    out_shapes = [
        jax.ShapeDtypeStruct((B, 1), f32),
        jax.ShapeDtypeStruct((B, 1), f32),
        jax.ShapeDtypeStruct((B, 1), f32),
    ]
    logp, ent, val = pl.pallas_call(
        _fused_body,
        grid=(nb,),
        in_specs=in_specs,
        out_specs=[row_spec(1), row_spec(1), row_spec(1)],
        out_shape=out_shapes,
        scratch_shapes=scratch,
    )(*ins)
    return action, logp[:, 0], ent[:, 0], val


def kernel(x, action_mask, phase_ids, action, params):
    return _run(x, action_mask, phase_ids, action, params)
